# initial kernel scaffold (unmeasured)
import jax
import jax.numpy as jnp
from jax import lax
from jax.experimental import pallas as pl
from jax.experimental.pallas import tpu as pltpu

N_DEV = 4


def kernel(x, w_mat, scale_x, scale_w):
    m, k_shard = x.shape
    _, n = w_mat.shape
    m_chunk = m // N_DEV

    def body(x_ref, w_ref, sx_ref, sw_ref, out_ref,
             comm_ref, rs_send_sems, rs_recv_sems, ag_send_sems, ag_recv_sems):
        my = lax.axis_index("i")
        left = (my - 1) % N_DEV
        right = (my + 1) % N_DEV

        barrier_sem = pltpu.get_barrier_semaphore()
        for nbr in (left, right):
            pl.semaphore_signal(
                barrier_sem, inc=1,
                device_id=(nbr,), device_id_type=pl.DeviceIdType.MESH,
            )
        pl.semaphore_wait(barrier_sem, 2)

        wb = w_ref[...].astype(jnp.bfloat16)
        for c in range(N_DEV):
            rows = pl.ds(c * m_chunk, m_chunk)
            out_ref[rows, :] = jnp.dot(
                x_ref[rows, :].astype(jnp.bfloat16), wb,
                preferred_element_type=jnp.float32,
            )

        def chunk_rows(idx):
            return pl.ds(idx * m_chunk, m_chunk)

        for s in range(N_DEV - 1):
            if s == 0:
                src = out_ref.at[chunk_rows((my - s) % N_DEV), :]
            else:
                src = comm_ref.at[s - 1]
            rdma = pltpu.make_async_remote_copy(
                src_ref=src,
                dst_ref=comm_ref.at[s],
                send_sem=rs_send_sems.at[s],
                recv_sem=rs_recv_sems.at[s],
                device_id=(right,),
                device_id_type=pl.DeviceIdType.MESH,
            )
            rdma.start()
            rdma.wait()
            recv_chunk = (my - s - 1) % N_DEV
            if s < N_DEV - 2:
                comm_ref[s] = comm_ref[s] + out_ref[chunk_rows(recv_chunk), :]
            else:
                scale = sx_ref[0, 0] * sw_ref[0, 0]
                reduced = comm_ref[s] + out_ref[chunk_rows(recv_chunk), :]
                out_ref[chunk_rows(recv_chunk), :] = jnp.maximum(
                    reduced * scale, 0.0)

        for t in range(N_DEV - 1):
            send_chunk = (my + 1 - t) % N_DEV
            recv_chunk = (my - t) % N_DEV
            rdma = pltpu.make_async_remote_copy(
                src_ref=out_ref.at[chunk_rows(send_chunk), :],
                dst_ref=out_ref.at[chunk_rows(recv_chunk), :],
                send_sem=ag_send_sems.at[t],
                recv_sem=ag_recv_sems.at[t],
                device_id=(right,),
                device_id_type=pl.DeviceIdType.MESH,
            )
            rdma.start()
            rdma.wait()

    return pl.pallas_call(
        body,
        out_shape=jax.ShapeDtypeStruct((m, n), jnp.float32),
        in_specs=[
            pl.BlockSpec(memory_space=pltpu.VMEM),
            pl.BlockSpec(memory_space=pltpu.VMEM),
            pl.BlockSpec(memory_space=pltpu.SMEM),
            pl.BlockSpec(memory_space=pltpu.SMEM),
        ],
        out_specs=pl.BlockSpec(memory_space=pltpu.VMEM),
        scratch_shapes=[
            pltpu.VMEM((N_DEV - 1, m_chunk, n), jnp.float32),
            pltpu.SemaphoreType.DMA((N_DEV - 1,)),
            pltpu.SemaphoreType.DMA((N_DEV - 1,)),
            pltpu.SemaphoreType.DMA((N_DEV - 1,)),
            pltpu.SemaphoreType.DMA((N_DEV - 1,)),
        ],
        compiler_params=pltpu.CompilerParams(collective_id=0),
    )(x, w_mat, scale_x.reshape(1, 1), scale_w.reshape(1, 1))


# baseline (device time: 635859 ns/iter reference)
import jax
import jax.numpy as jnp
from jax import lax
from jax.experimental import pallas as pl
from jax.experimental.pallas import tpu as pltpu

N_DEV = 4


def kernel(x, w_mat, scale_x, scale_w):
    m, k_shard = x.shape
    _, n = w_mat.shape
    mc = m // N_DEV

    def body(x_hbm, w_ref, sx_ref, sw_ref, out_hbm,
             xbuf, wb, stage, comm,
             load_sem, store_sem,
             rs_send, rs_recv, ag_send, ag_recv):
        my = lax.axis_index("i")
        left = (my - 1) % N_DEV
        right = (my + 1) % N_DEV

        def rows(idx):
            return pl.ds(idx * mc, mc)

        barrier_sem = pltpu.get_barrier_semaphore()
        for nbr in (left, right):
            pl.semaphore_signal(
                barrier_sem, inc=1,
                device_id=(nbr,), device_id_type=pl.DeviceIdType.MESH,
            )
        pl.semaphore_wait(barrier_sem, 2)

        wb[...] = w_ref[...].astype(jnp.bfloat16)
        for c in range(N_DEV):
            cp = pltpu.make_async_copy(x_hbm.at[rows(c), :], xbuf, load_sem)
            cp.start()
            cp.wait()
            stage[...] = jnp.dot(
                xbuf[...].astype(jnp.bfloat16), wb[...],
                preferred_element_type=jnp.float32,
            )
            st = pltpu.make_async_copy(stage, out_hbm.at[rows(c), :], store_sem)
            st.start()
            st.wait()

        scale = sx_ref[0, 0] * sw_ref[0, 0]

        for s in range(N_DEV - 1):
            if s == 0:
                src = out_hbm.at[rows((my - s) % N_DEV), :]
            else:
                src = comm.at[(s - 1) % 2]
            rdma = pltpu.make_async_remote_copy(
                src_ref=src,
                dst_ref=comm.at[s % 2],
                send_sem=rs_send.at[s],
                recv_sem=rs_recv.at[s],
                device_id=(right,),
                device_id_type=pl.DeviceIdType.MESH,
            )
            rdma.start()
            pc = (my - s - 1) % N_DEV
            cp = pltpu.make_async_copy(out_hbm.at[rows(pc), :], stage, load_sem)
            cp.start()
            cp.wait()
            rdma.wait()
            if s < N_DEV - 2:
                comm[s % 2] = comm[s % 2] + stage[...]
            else:
                stage[...] = jnp.maximum(
                    (comm[s % 2] + stage[...]) * scale, 0.0)
                st = pltpu.make_async_copy(
                    stage, out_hbm.at[rows(pc), :], store_sem)
                st.start()
                st.wait()

        for t in range(N_DEV - 1):
            recv_slot = (t + 1) % 2
            src = stage if t == 0 else comm.at[t % 2]
            rdma = pltpu.make_async_remote_copy(
                src_ref=src,
                dst_ref=comm.at[recv_slot],
                send_sem=ag_send.at[t],
                recv_sem=ag_recv.at[t],
                device_id=(right,),
                device_id_type=pl.DeviceIdType.MESH,
            )
            rdma.start()
            rdma.wait()
            rc = (my - t) % N_DEV
            st = pltpu.make_async_copy(
                comm.at[recv_slot], out_hbm.at[rows(rc), :], store_sem)
            st.start()
            st.wait()

    return pl.pallas_call(
        body,
        out_shape=jax.ShapeDtypeStruct((m, n), jnp.float32),
        in_specs=[
            pl.BlockSpec(memory_space=pl.ANY),
            pl.BlockSpec(memory_space=pltpu.VMEM),
            pl.BlockSpec(memory_space=pltpu.SMEM),
            pl.BlockSpec(memory_space=pltpu.SMEM),
        ],
        out_specs=pl.BlockSpec(memory_space=pl.ANY),
        scratch_shapes=[
            pltpu.VMEM((mc, k_shard), x.dtype),
            pltpu.VMEM((k_shard, n), jnp.bfloat16),
            pltpu.VMEM((mc, n), jnp.float32),
            pltpu.VMEM((2, mc, n), jnp.float32),
            pltpu.SemaphoreType.DMA,
            pltpu.SemaphoreType.DMA,
            pltpu.SemaphoreType.DMA((N_DEV - 1,)),
            pltpu.SemaphoreType.DMA((N_DEV - 1,)),
            pltpu.SemaphoreType.DMA((N_DEV - 1,)),
            pltpu.SemaphoreType.DMA((N_DEV - 1,)),
        ],
        compiler_params=pltpu.CompilerParams(
            collective_id=0, vmem_limit_bytes=56 * 1024 * 1024),
    )(x, w_mat, scale_x.reshape(1, 1), scale_w.reshape(1, 1))


# device time: 326532 ns/iter; 1.9473x vs baseline; 1.9473x over previous
import jax
import jax.numpy as jnp
from jax import lax
from jax.experimental import pallas as pl
from jax.experimental.pallas import tpu as pltpu

N_DEV = 4
W_SLICES = 4


def kernel(x, w_mat, scale_x, scale_w):
    m, k_shard = x.shape
    _, n = w_mat.shape
    mc = m // N_DEV
    nh = n // 2
    kw = k_shard // W_SLICES

    def body(x_hbm, w_hbm, sx_ref, sw_ref, out_hbm,
             wtmp, wb, xbuf, stage, parts, commR, commL,
             load_sem, store_sems,
             rsR_send, rsR_recv, rsL_send, rsL_recv,
             agR_send, agR_recv, agL_send, agL_recv,
             creditR, creditL):
        my = lax.axis_index("i")
        left = (my - 1) % N_DEV
        right = (my + 1) % N_DEV

        def rows(idx):
            return pl.ds(idx * mc, mc)

        def rdma(src, dst, ssem, rsem, dev):
            return pltpu.make_async_remote_copy(
                src_ref=src, dst_ref=dst, send_sem=ssem, recv_sem=rsem,
                device_id=(dev,), device_id_type=pl.DeviceIdType.MESH)

        def credit_signal():
            pl.semaphore_signal(creditR, inc=1, device_id=(left,),
                                device_id_type=pl.DeviceIdType.MESH)
            pl.semaphore_signal(creditL, inc=1, device_id=(right,),
                                device_id_type=pl.DeviceIdType.MESH)

        def credit_wait():
            pl.semaphore_wait(creditR, 1)
            pl.semaphore_wait(creditL, 1)

        barrier_sem = pltpu.get_barrier_semaphore()
        for nbr in (left, right):
            pl.semaphore_signal(
                barrier_sem, inc=1,
                device_id=(nbr,), device_id_type=pl.DeviceIdType.MESH,
            )
        pl.semaphore_wait(barrier_sem, 2)

        for j in range(W_SLICES):
            cp = pltpu.make_async_copy(
                w_hbm.at[pl.ds(j * kw, kw), :], wtmp, load_sem)
            cp.start()
            cp.wait()
            wb[pl.ds(j * kw, kw), :] = wtmp[...].astype(jnp.bfloat16)

        cp = pltpu.make_async_copy(x_hbm.at[rows(my), :], xbuf, load_sem)
        cp.start()
        cp.wait()
        stage[...] = jnp.dot(
            xbuf[...].astype(jnp.bfloat16), wb[...],
            preferred_element_type=jnp.float32)

        r0 = rdma(stage.at[:, pl.ds(0, nh)], commR.at[0],
                  rsR_send.at[0], rsR_recv.at[0], right)
        l0 = rdma(stage.at[:, pl.ds(nh, nh)], commL.at[0],
                  rsL_send.at[0], rsL_recv.at[0], left)
        r0.start()
        l0.start()

        for j, c in enumerate(((my - 1) % N_DEV, (my + 1) % N_DEV,
                               (my + 2) % N_DEV)):
            cp = pltpu.make_async_copy(x_hbm.at[rows(c), :], xbuf, load_sem)
            cp.start()
            cp.wait()
            parts[j, :, :] = jnp.dot(
                xbuf[...].astype(jnp.bfloat16), wb[...],
                preferred_element_type=jnp.float32).astype(jnp.bfloat16)

        r0.wait()
        l0.wait()
        commR[0, :, :] = commR[0, :, :] + parts[0, :, :nh].astype(jnp.float32)
        commL[0, :, :] = commL[0, :, :] + parts[1, :, nh:].astype(jnp.float32)

        r1 = rdma(commR.at[0], commR.at[1],
                  rsR_send.at[1], rsR_recv.at[1], right)
        l1 = rdma(commL.at[0], commL.at[1],
                  rsL_send.at[1], rsL_recv.at[1], left)
        r1.start()
        l1.start()
        r1.wait()
        l1.wait()
        credit_signal()
        commR[1, :, :] = commR[1, :, :] + parts[2, :, :nh].astype(jnp.float32)
        commL[1, :, :] = commL[1, :, :] + parts[2, :, nh:].astype(jnp.float32)

        credit_wait()
        r2 = rdma(commR.at[1], commR.at[0],
                  rsR_send.at[2], rsR_recv.at[2], right)
        l2 = rdma(commL.at[1], commL.at[0],
                  rsL_send.at[2], rsL_recv.at[2], left)
        r2.start()
        l2.start()
        r2.wait()
        l2.wait()
        credit_signal()

        scale = sx_ref[0, 0] * sw_ref[0, 0]
        stage[:, 0:nh] = jnp.maximum(
            (commR[0, :, :] + parts[1, :, :nh].astype(jnp.float32)) * scale, 0.0)
        stage[:, nh:n] = jnp.maximum(
            (commL[0, :, :] + parts[0, :, nh:].astype(jnp.float32)) * scale, 0.0)
        credit_signal()
        st0 = pltpu.make_async_copy(
            stage.at[:, pl.ds(0, nh)],
            out_hbm.at[rows((my + 1) % N_DEV), pl.ds(0, nh)], store_sems.at[0])
        st1 = pltpu.make_async_copy(
            stage.at[:, pl.ds(nh, nh)],
            out_hbm.at[rows((my - 1) % N_DEV), pl.ds(nh, nh)], store_sems.at[1])
        st0.start()
        st1.start()

        credit_wait()
        a0r = rdma(stage.at[:, pl.ds(0, nh)], commR.at[1],
                   agR_send.at[0], agR_recv.at[0], right)
        a0l = rdma(stage.at[:, pl.ds(nh, nh)], commL.at[1],
                   agL_send.at[0], agL_recv.at[0], left)
        a0r.start()
        a0l.start()
        st0.wait()
        st1.wait()
        a0r.wait()
        a0l.wait()

        credit_wait()
        a1r = rdma(commR.at[1], commR.at[0],
                   agR_send.at[1], agR_recv.at[1], right)
        a1l = rdma(commL.at[1], commL.at[0],
                   agL_send.at[1], agL_recv.at[1], left)
        a1r.start()
        a1l.start()
        st0 = pltpu.make_async_copy(
            commR.at[1], out_hbm.at[rows(my), pl.ds(0, nh)], store_sems.at[0])
        st1 = pltpu.make_async_copy(
            commL.at[1], out_hbm.at[rows(my), pl.ds(nh, nh)], store_sems.at[1])
        st0.start()
        st1.start()
        st0.wait()
        st1.wait()
        a1r.wait()
        a1l.wait()
        credit_signal()

        credit_wait()
        a2r = rdma(commR.at[0], commR.at[1],
                   agR_send.at[2], agR_recv.at[2], right)
        a2l = rdma(commL.at[0], commL.at[1],
                   agL_send.at[2], agL_recv.at[2], left)
        a2r.start()
        a2l.start()
        st0 = pltpu.make_async_copy(
            commR.at[0], out_hbm.at[rows((my - 1) % N_DEV), pl.ds(0, nh)],
            store_sems.at[0])
        st1 = pltpu.make_async_copy(
            commL.at[0], out_hbm.at[rows((my + 1) % N_DEV), pl.ds(nh, nh)],
            store_sems.at[1])
        st0.start()
        st1.start()
        st0.wait()
        st1.wait()
        a2r.wait()
        a2l.wait()
        st0 = pltpu.make_async_copy(
            commR.at[1], out_hbm.at[rows((my + 2) % N_DEV), pl.ds(0, nh)],
            store_sems.at[0])
        st1 = pltpu.make_async_copy(
            commL.at[1], out_hbm.at[rows((my + 2) % N_DEV), pl.ds(nh, nh)],
            store_sems.at[1])
        st0.start()
        st1.start()
        st0.wait()
        st1.wait()

    return pl.pallas_call(
        body,
        out_shape=jax.ShapeDtypeStruct((m, n), jnp.float32),
        in_specs=[
            pl.BlockSpec(memory_space=pl.ANY),
            pl.BlockSpec(memory_space=pl.ANY),
            pl.BlockSpec(memory_space=pltpu.SMEM),
            pl.BlockSpec(memory_space=pltpu.SMEM),
        ],
        out_specs=pl.BlockSpec(memory_space=pl.ANY),
        scratch_shapes=[
            pltpu.VMEM((kw, n), jnp.float32),
            pltpu.VMEM((k_shard, n), jnp.bfloat16),
            pltpu.VMEM((mc, k_shard), jnp.float32),
            pltpu.VMEM((mc, n), jnp.float32),
            pltpu.VMEM((3, mc, n), jnp.bfloat16),
            pltpu.VMEM((2, mc, nh), jnp.float32),
            pltpu.VMEM((2, mc, nh), jnp.float32),
            pltpu.SemaphoreType.DMA,
            pltpu.SemaphoreType.DMA((2,)),
            pltpu.SemaphoreType.DMA((3,)),
            pltpu.SemaphoreType.DMA((3,)),
            pltpu.SemaphoreType.DMA((3,)),
            pltpu.SemaphoreType.DMA((3,)),
            pltpu.SemaphoreType.DMA((3,)),
            pltpu.SemaphoreType.DMA((3,)),
            pltpu.SemaphoreType.DMA((3,)),
            pltpu.SemaphoreType.DMA((3,)),
            pltpu.SemaphoreType.REGULAR,
            pltpu.SemaphoreType.REGULAR,
        ],
        compiler_params=pltpu.CompilerParams(
            collective_id=0, vmem_limit_bytes=60 * 1024 * 1024),
    )(x, w_mat, scale_x.reshape(1, 1), scale_w.reshape(1, 1))
